# Initial kernel scaffold; baseline (speedup 1.0000x reference)
import jax
import jax.numpy as jnp
from jax.experimental import pallas as pl


def kernel(s0, s1, x_0, edge_index, weights, W0, gamma0, beta0, W1, gamma1, beta1, Wg, Wss, Wsn):
    raise NotImplementedError("write your pallas kernel here")


# speedup vs baseline: 5.6121x; 5.6121x over previous
import functools

import jax
import jax.numpy as jnp
from jax import lax
from jax.experimental import pallas as pl
from jax.experimental.pallas import tpu as pltpu
from jax.experimental.pallas import tpu_sc as plsc

_N = 10000
_C = 128
_E = 320000
_ALPHA = 0.1
_K = 128
_NCH = _E // _K
_NSC = 2
_NTILE = 16
_CH_PER_SC = _NCH // _NSC
_SPAN = 624
_WB = 104
_TAIL_ROW = 16 * _SPAN


def _chunk_range(cid, sid):
    lo = cid * _CH_PER_SC + (_CH_PER_SC * sid) // _NTILE
    hi = cid * _CH_PER_SC + (_CH_PER_SC * (sid + 1)) // _NTILE
    return lo, hi


def _zero_acc(acc, zbuf, sid):
    row0 = sid * _SPAN
    for kk in range(_SPAN // _WB):
        pltpu.sync_copy(zbuf, acc.at[pl.ds(row0 + kk * _WB, _WB)])

    @pl.when(sid == _NTILE - 1)
    def _zero_tail():
        pltpu.sync_copy(zbuf.at[pl.ds(0, 16)], acc.at[pl.ds(_TAIL_ROW, 16)])


def _writeback(acc, zbuf, out_hbm, cid, sid):
    row0 = sid * _SPAN
    for kk in range(_SPAN // _WB):
        r = row0 + kk * _WB
        pltpu.sync_copy(acc.at[pl.ds(r, _WB)], zbuf)
        pltpu.sync_copy(zbuf, out_hbm.at[cid, pl.ds(r, _WB)])

    @pl.when(sid == _NTILE - 1)
    def _wb_tail():
        pltpu.sync_copy(acc.at[pl.ds(_TAIL_ROW, 16)], zbuf.at[pl.ds(0, 16)])
        pltpu.sync_copy(zbuf.at[pl.ds(0, 16)],
                        out_hbm.at[cid, pl.ds(_TAIL_ROW, 16)])


def _sc_agg_body(x_hbm, src_hbm, dst_hbm, zeros_hbm,
                 agg_out, acc, buf0, zbuf, idx_s, idx_d, sem):
    cid = lax.axis_index("c")
    sid = lax.axis_index("s")

    pltpu.sync_copy(zeros_hbm, zbuf)
    _zero_acc(acc, zbuf, sid)
    plsc.subcore_barrier()

    lo, hi = _chunk_range(cid, sid)

    def chunk_step(j, carry):
        pltpu.sync_copy(src_hbm.at[j], idx_s)
        pltpu.sync_copy(dst_hbm.at[j], idx_d)
        pltpu.async_copy(x_hbm.at[idx_s.at[0]], buf0, sem).wait()
        pltpu.sync_copy(buf0, acc.at[idx_d.at[0]], add=True)
        return carry

    lax.fori_loop(lo, hi, chunk_step, 0)
    plsc.subcore_barrier()
    _writeback(acc, zbuf, agg_out, cid, sid)


_sc_agg = pl.kernel(
    _sc_agg_body,
    out_type=jax.ShapeDtypeStruct((_NSC, _N, _C), jnp.float32),
    mesh=plsc.VectorSubcoreMesh(core_axis_name="c", subcore_axis_name="s"),
    scratch_types=[
        pltpu.VMEM_SHARED((_N, _C), jnp.float32),
        pltpu.VMEM((_K, _C), jnp.float32),
        pltpu.VMEM((_WB, _C), jnp.float32),
        pltpu.VMEM((1, _K), jnp.int32),
        pltpu.VMEM((1, _K), jnp.int32),
        pltpu.SemaphoreType.DMA,
    ],
)


def _sc_deg_body(dst_hbm, zeros_hbm, ones_hbm,
                 deg_out, acc, ones_v, zbuf, idx_d, sem):
    cid = lax.axis_index("c")
    sid = lax.axis_index("s")

    pltpu.sync_copy(zeros_hbm, zbuf)
    pltpu.sync_copy(ones_hbm, ones_v)
    _zero_acc(acc, zbuf, sid)
    plsc.subcore_barrier()

    lo, hi = _chunk_range(cid, sid)

    def chunk_step(j, carry):
        pltpu.sync_copy(dst_hbm.at[j], idx_d)
        pltpu.sync_copy(ones_v, acc.at[idx_d.at[0]], add=True)
        return carry

    lax.fori_loop(lo, hi, chunk_step, 0)
    plsc.subcore_barrier()
    _writeback(acc, zbuf, deg_out, cid, sid)


_sc_deg = pl.kernel(
    _sc_deg_body,
    out_type=jax.ShapeDtypeStruct((_NSC, _N, _C), jnp.float32),
    mesh=plsc.VectorSubcoreMesh(core_axis_name="c", subcore_axis_name="s"),
    scratch_types=[
        pltpu.VMEM_SHARED((_N, _C), jnp.float32),
        pltpu.VMEM((_K, _C), jnp.float32),
        pltpu.VMEM((_WB, _C), jnp.float32),
        pltpu.VMEM((1, _K), jnp.int32),
        pltpu.SemaphoreType.DMA,
    ],
)


def _mlp_body(x_ref, w_ref, g_ref, b_ref, o_ref):
    h = jnp.dot(x_ref[...], w_ref[...], preferred_element_type=jnp.float32)
    mu = jnp.mean(h, axis=0, keepdims=True)
    var = jnp.mean((h - mu) ** 2, axis=0, keepdims=True)
    hn = (h - mu) * lax.rsqrt(var + 1e-5)
    o_ref[...] = jnp.maximum(g_ref[...] * hn + b_ref[...], 0.0)


def _tc_mlp(x, w, gamma, beta):
    return pl.pallas_call(
        _mlp_body,
        out_shape=jax.ShapeDtypeStruct((_N, _C), jnp.float32),
    )(x, w, gamma.reshape(1, _C), beta.reshape(1, _C))


_BN = 1000


def _fin0_body(pdeg_ref, pa0_ref, pa1_ref, inv_ref, a0_ref, a1_ref):
    deg = pdeg_ref[0] + pdeg_ref[1]
    inv = 1.0 / jnp.maximum(deg, 1.0)
    inv_ref[...] = inv[:, 0:1]
    a0_ref[...] = (pa0_ref[0] + pa0_ref[1]) * inv
    a1_ref[...] = (pa1_ref[0] + pa1_ref[1]) * inv


def _tc_finalize0(pdeg, pa0, pa1):
    grid = (_N // _BN,)
    bs_p = pl.BlockSpec((_NSC, _BN, _C), lambda i: (0, i, 0))
    bs_o = pl.BlockSpec((_BN, _C), lambda i: (i, 0))
    bs_i = pl.BlockSpec((_BN, 1), lambda i: (i, 0))
    return pl.pallas_call(
        _fin0_body,
        grid=grid,
        in_specs=[bs_p, bs_p, bs_p],
        out_specs=[bs_i, bs_o, bs_o],
        out_shape=[
            jax.ShapeDtypeStruct((_N, 1), jnp.float32),
            jax.ShapeDtypeStruct((_N, _C), jnp.float32),
            jax.ShapeDtypeStruct((_N, _C), jnp.float32),
        ],
    )(pdeg, pa0, pa1)


def _fin_body(pa_ref, inv_ref, a_ref):
    a_ref[...] = (pa_ref[0] + pa_ref[1]) * inv_ref[...]


def _tc_finalize(pa, inv_deg):
    grid = (_N // _BN,)
    return pl.pallas_call(
        _fin_body,
        grid=grid,
        in_specs=[
            pl.BlockSpec((_NSC, _BN, _C), lambda i: (0, i, 0)),
            pl.BlockSpec((_BN, 1), lambda i: (i, 0)),
        ],
        out_specs=pl.BlockSpec((_BN, _C), lambda i: (i, 0)),
        out_shape=jax.ShapeDtypeStruct((_N, _C), jnp.float32),
    )(pa, inv_deg)


def _combine_body(k, wrow_ref, x0_ref, *refs):
    hs = refs[:k]
    asm = refs[k:2 * k]
    wg_ref, wss_ref, wsn_ref, o_ref = refs[2 * k:]
    x0 = x0_ref[...]
    acc = jnp.zeros_like(x0)
    for j in range(k):
        h = hs[j][...]
        a = asm[j][...]
        w1 = wrow_ref[j, 1]
        w2 = wrow_ref[j, 2]
        w3 = wrow_ref[j, 3]
        u = (1.0 - _ALPHA) * a + _ALPHA * x0
        gcn = jnp.maximum(
            jnp.dot(u, wg_ref[j], preferred_element_type=jnp.float32), 0.0)
        sage = jnp.maximum(
            jnp.dot(h, wss_ref[j], preferred_element_type=jnp.float32)
            + jnp.dot(a, wsn_ref[j], preferred_element_type=jnp.float32), 0.0)
        acc = acc + w1 * h + w2 * gcn + w3 * sage
    o_ref[...] = acc


def _tc_combine(k, wrow, x0, hs, ams, wg, wss, wsn):
    grid = (_N // _BN,)
    bs_row = pl.BlockSpec((_BN, _C), lambda i: (i, 0))
    bs_w = pl.BlockSpec((k, _C, _C), lambda i: (0, 0, 0))
    return pl.pallas_call(
        functools.partial(_combine_body, k),
        grid=grid,
        in_specs=([pl.BlockSpec(memory_space=pltpu.SMEM), bs_row]
                  + [bs_row] * (2 * k) + [bs_w, bs_w, bs_w]),
        out_specs=bs_row,
        out_shape=jax.ShapeDtypeStruct((_N, _C), jnp.float32),
    )(wrow, x0, *hs, *ams, wg, wss, wsn)


def kernel(s0, s1, x_0, edge_index, weights, W0, gamma0, beta0,
           W1, gamma1, beta1, Wg, Wss, Wsn):
    src3d = edge_index[0].astype(jnp.int32).reshape(_NCH, 1, _K)
    dst3d = edge_index[1].astype(jnp.int32).reshape(_NCH, 1, _K)
    zeros_c = jnp.zeros((_WB, _C), jnp.float32)
    ones_c = jnp.ones((_K, _C), jnp.float32)

    m0 = _tc_mlp(s0, W0, gamma0, beta0)
    m1 = _tc_mlp(s1, W1, gamma1, beta1)

    def agg(x):
        return _sc_agg(x, src3d, dst3d, zeros_c)

    pdeg = _sc_deg(dst3d, zeros_c, ones_c)
    pa0 = agg(m0)
    pa1 = agg(m1)
    inv_deg, a0, a1 = _tc_finalize0(pdeg, pa0, pa1)

    states = [m0, m1]
    aggs = [a0, a1]
    offset = 0
    for step in range(4):
        k = len(states)
        sl = slice(offset, offset + k)
        new_state = _tc_combine(k, weights[sl], x_0, states, aggs,
                                Wg[sl], Wss[sl], Wsn[sl])
        offset += k
        states.append(new_state)
        if step < 3:
            aggs.append(_tc_finalize(agg(new_state), inv_deg))

    return jnp.concatenate(states[-4:], axis=1)


# speedup vs baseline: 10.2140x; 1.8200x over previous
import functools

import jax
import jax.numpy as jnp
from jax import lax
from jax.experimental import pallas as pl
from jax.experimental.pallas import tpu as pltpu
from jax.experimental.pallas import tpu_sc as plsc

_N = 10000
_C = 128
_E = 320000
_ALPHA = 0.1
_K = 125
_G = 8
_NGRP = _E // (_K * _G)
_NSC = 2
_NTILE = 16
_GRP_PER_TILE = _NGRP // (_NSC * _NTILE)
_SPAN = 624
_WB = 104
_TAIL_ROW = 16 * _SPAN


def _zero_acc(acc, zbuf, sid):
    row0 = sid * _SPAN
    for kk in range(_SPAN // _WB):
        pltpu.sync_copy(zbuf, acc.at[pl.ds(row0 + kk * _WB, _WB)])

    @pl.when(sid == _NTILE - 1)
    def _zero_tail():
        pltpu.sync_copy(zbuf.at[pl.ds(0, 16)], acc.at[pl.ds(_TAIL_ROW, 16)])


def _writeback(acc, zbuf, out_hbm, cid, sid):
    row0 = sid * _SPAN
    for kk in range(_SPAN // _WB):
        r = row0 + kk * _WB
        pltpu.sync_copy(acc.at[pl.ds(r, _WB)], zbuf)
        pltpu.sync_copy(zbuf, out_hbm.at[cid, pl.ds(r, _WB)])

    @pl.when(sid == _NTILE - 1)
    def _wb_tail():
        pltpu.sync_copy(acc.at[pl.ds(_TAIL_ROW, 16)], zbuf.at[pl.ds(0, 16)])
        pltpu.sync_copy(zbuf.at[pl.ds(0, 16)],
                        out_hbm.at[cid, pl.ds(_TAIL_ROW, 16)])


def _sc_agg_body(x_hbm, src_hbm, dst_hbm, zeros_hbm,
                 agg_out, acc, buf, isrc, idst, zbuf,
                 gsem0, gsem1, isem_s, isem_d):
    cid = lax.axis_index("c")
    sid = lax.axis_index("s")

    pltpu.sync_copy(zeros_hbm, zbuf)
    _zero_acc(acc, zbuf, sid)
    plsc.subcore_barrier()

    g0 = (cid * _NTILE + sid) * _GRP_PER_TILE
    gsems = (gsem0, gsem1)

    pltpu.sync_copy(src_hbm.at[g0], isrc.at[0])
    pltpu.sync_copy(dst_hbm.at[g0], idst.at[0])

    def group_step(gi, carry):
        slot = gi % 2
        pslot = 1 - slot

        @pl.when(gi > 0)
        def _wait_idx():
            pltpu.make_async_copy(src_hbm.at[g0], isrc.at[slot], isem_s).wait()
            pltpu.make_async_copy(dst_hbm.at[g0], idst.at[slot], isem_d).wait()

        d_prev = pltpu.async_copy(
            x_hbm.at[isrc.at[slot, 0]], buf.at[0], gsems[0])

        @pl.when(gi > 0)
        def _straddle_scatter():
            pltpu.sync_copy(buf.at[1], acc.at[idst.at[pslot, _G - 1]],
                            add=True)

        for c in range(1, _G):
            d_c = pltpu.async_copy(
                x_hbm.at[isrc.at[slot, c]], buf.at[c % 2], gsems[c % 2])
            d_prev.wait()
            pltpu.sync_copy(buf.at[(c - 1) % 2],
                            acc.at[idst.at[slot, c - 1]], add=True)
            if c == 1:
                @pl.when(gi < _GRP_PER_TILE - 1)
                def _prefetch_idx():
                    pltpu.async_copy(src_hbm.at[g0 + gi + 1],
                                     isrc.at[pslot], isem_s)
                    pltpu.async_copy(dst_hbm.at[g0 + gi + 1],
                                     idst.at[pslot], isem_d)
            d_prev = d_c
        d_prev.wait()
        return carry

    lax.fori_loop(0, _GRP_PER_TILE, group_step, 0)
    last_slot = (_GRP_PER_TILE - 1) % 2
    pltpu.sync_copy(buf.at[1], acc.at[idst.at[last_slot, _G - 1]], add=True)

    plsc.subcore_barrier()
    _writeback(acc, zbuf, agg_out, cid, sid)


_sc_agg = pl.kernel(
    _sc_agg_body,
    out_type=jax.ShapeDtypeStruct((_NSC, _N, _C), jnp.float32),
    mesh=plsc.VectorSubcoreMesh(core_axis_name="c", subcore_axis_name="s"),
    scratch_types=[
        pltpu.VMEM_SHARED((_N, _C), jnp.float32),
        pltpu.VMEM((2, _K, _C), jnp.float32),
        pltpu.VMEM((2, _G, _K), jnp.int32),
        pltpu.VMEM((2, _G, _K), jnp.int32),
        pltpu.VMEM((_WB, _C), jnp.float32),
        pltpu.SemaphoreType.DMA,
        pltpu.SemaphoreType.DMA,
        pltpu.SemaphoreType.DMA,
        pltpu.SemaphoreType.DMA,
    ],
)


def _sc_deg_body(dst_hbm, zeros_hbm, ones_hbm,
                 deg_out, acc, ones_v, zbuf, idst, isem_d):
    cid = lax.axis_index("c")
    sid = lax.axis_index("s")

    pltpu.sync_copy(zeros_hbm, zbuf)
    pltpu.sync_copy(ones_hbm, ones_v)
    _zero_acc(acc, zbuf, sid)
    plsc.subcore_barrier()

    g0 = (cid * _NTILE + sid) * _GRP_PER_TILE
    pltpu.sync_copy(dst_hbm.at[g0], idst.at[0])

    def group_step(gi, carry):
        slot = gi % 2
        pslot = 1 - slot

        @pl.when(gi > 0)
        def _wait_idx():
            pltpu.make_async_copy(dst_hbm.at[g0], idst.at[slot], isem_d).wait()

        @pl.when(gi < _GRP_PER_TILE - 1)
        def _prefetch_idx():
            pltpu.async_copy(dst_hbm.at[g0 + gi + 1], idst.at[pslot], isem_d)

        for c in range(_G):
            pltpu.sync_copy(ones_v, acc.at[idst.at[slot, c]], add=True)
        return carry

    lax.fori_loop(0, _GRP_PER_TILE, group_step, 0)
    plsc.subcore_barrier()
    _writeback(acc, zbuf, deg_out, cid, sid)


_sc_deg = pl.kernel(
    _sc_deg_body,
    out_type=jax.ShapeDtypeStruct((_NSC, _N, _C), jnp.float32),
    mesh=plsc.VectorSubcoreMesh(core_axis_name="c", subcore_axis_name="s"),
    scratch_types=[
        pltpu.VMEM_SHARED((_N, _C), jnp.float32),
        pltpu.VMEM((_K, _C), jnp.float32),
        pltpu.VMEM((_WB, _C), jnp.float32),
        pltpu.VMEM((2, _G, _K), jnp.int32),
        pltpu.SemaphoreType.DMA,
    ],
)


def _mlp_body(x_ref, w_ref, g_ref, b_ref, o_ref):
    h = jnp.dot(x_ref[...], w_ref[...], preferred_element_type=jnp.float32)
    mu = jnp.mean(h, axis=0, keepdims=True)
    var = jnp.mean((h - mu) ** 2, axis=0, keepdims=True)
    hn = (h - mu) * lax.rsqrt(var + 1e-5)
    o_ref[...] = jnp.maximum(g_ref[...] * hn + b_ref[...], 0.0)


def _tc_mlp(x, w, gamma, beta):
    return pl.pallas_call(
        _mlp_body,
        out_shape=jax.ShapeDtypeStruct((_N, _C), jnp.float32),
    )(x, w, gamma.reshape(1, _C), beta.reshape(1, _C))


_BN = 1000


def _fin0_body(pdeg_ref, pa0_ref, pa1_ref, inv_ref, a0_ref, a1_ref):
    deg = pdeg_ref[0] + pdeg_ref[1]
    inv = 1.0 / jnp.maximum(deg, 1.0)
    inv_ref[...] = inv[:, 0:1]
    a0_ref[...] = (pa0_ref[0] + pa0_ref[1]) * inv
    a1_ref[...] = (pa1_ref[0] + pa1_ref[1]) * inv


def _tc_finalize0(pdeg, pa0, pa1):
    grid = (_N // _BN,)
    bs_p = pl.BlockSpec((_NSC, _BN, _C), lambda i: (0, i, 0))
    bs_o = pl.BlockSpec((_BN, _C), lambda i: (i, 0))
    bs_i = pl.BlockSpec((_BN, 1), lambda i: (i, 0))
    return pl.pallas_call(
        _fin0_body,
        grid=grid,
        in_specs=[bs_p, bs_p, bs_p],
        out_specs=[bs_i, bs_o, bs_o],
        out_shape=[
            jax.ShapeDtypeStruct((_N, 1), jnp.float32),
            jax.ShapeDtypeStruct((_N, _C), jnp.float32),
            jax.ShapeDtypeStruct((_N, _C), jnp.float32),
        ],
    )(pdeg, pa0, pa1)


def _fin_body(pa_ref, inv_ref, a_ref):
    a_ref[...] = (pa_ref[0] + pa_ref[1]) * inv_ref[...]


def _tc_finalize(pa, inv_deg):
    grid = (_N // _BN,)
    return pl.pallas_call(
        _fin_body,
        grid=grid,
        in_specs=[
            pl.BlockSpec((_NSC, _BN, _C), lambda i: (0, i, 0)),
            pl.BlockSpec((_BN, 1), lambda i: (i, 0)),
        ],
        out_specs=pl.BlockSpec((_BN, _C), lambda i: (i, 0)),
        out_shape=jax.ShapeDtypeStruct((_N, _C), jnp.float32),
    )(pa, inv_deg)


def _combine_body(k, wrow_ref, x0_ref, *refs):
    hs = refs[:k]
    asm = refs[k:2 * k]
    wg_ref, wss_ref, wsn_ref, o_ref = refs[2 * k:]
    x0 = x0_ref[...]
    acc = jnp.zeros_like(x0)
    for j in range(k):
        h = hs[j][...]
        a = asm[j][...]
        w1 = wrow_ref[j, 1]
        w2 = wrow_ref[j, 2]
        w3 = wrow_ref[j, 3]
        u = (1.0 - _ALPHA) * a + _ALPHA * x0
        gcn = jnp.maximum(
            jnp.dot(u, wg_ref[j], preferred_element_type=jnp.float32), 0.0)
        sage = jnp.maximum(
            jnp.dot(h, wss_ref[j], preferred_element_type=jnp.float32)
            + jnp.dot(a, wsn_ref[j], preferred_element_type=jnp.float32), 0.0)
        acc = acc + w1 * h + w2 * gcn + w3 * sage
    o_ref[...] = acc


def _tc_combine(k, wrow, x0, hs, ams, wg, wss, wsn):
    grid = (_N // _BN,)
    bs_row = pl.BlockSpec((_BN, _C), lambda i: (i, 0))
    bs_w = pl.BlockSpec((k, _C, _C), lambda i: (0, 0, 0))
    return pl.pallas_call(
        functools.partial(_combine_body, k),
        grid=grid,
        in_specs=([pl.BlockSpec(memory_space=pltpu.SMEM), bs_row]
                  + [bs_row] * (2 * k) + [bs_w, bs_w, bs_w]),
        out_specs=bs_row,
        out_shape=jax.ShapeDtypeStruct((_N, _C), jnp.float32),
    )(wrow, x0, *hs, *ams, wg, wss, wsn)


def kernel(s0, s1, x_0, edge_index, weights, W0, gamma0, beta0,
           W1, gamma1, beta1, Wg, Wss, Wsn):
    src3d = edge_index[0].astype(jnp.int32).reshape(_NGRP, _G, _K)
    dst3d = edge_index[1].astype(jnp.int32).reshape(_NGRP, _G, _K)
    zeros_c = jnp.zeros((_WB, _C), jnp.float32)
    ones_c = jnp.ones((_K, _C), jnp.float32)

    m0 = _tc_mlp(s0, W0, gamma0, beta0)
    m1 = _tc_mlp(s1, W1, gamma1, beta1)

    def agg(x):
        return _sc_agg(x, src3d, dst3d, zeros_c)

    pdeg = _sc_deg(dst3d, zeros_c, ones_c)
    pa0 = agg(m0)
    pa1 = agg(m1)
    inv_deg, a0, a1 = _tc_finalize0(pdeg, pa0, pa1)

    states = [m0, m1]
    aggs = [a0, a1]
    offset = 0
    for step in range(4):
        k = len(states)
        sl = slice(offset, offset + k)
        new_state = _tc_combine(k, weights[sl], x_0, states, aggs,
                                Wg[sl], Wss[sl], Wsn[sl])
        offset += k
        states.append(new_state)
        if step < 3:
            aggs.append(_tc_finalize(agg(new_state), inv_deg))

    return jnp.concatenate(states[-4:], axis=1)


# speedup vs baseline: 10.3919x; 1.0174x over previous
import functools

import jax
import jax.numpy as jnp
from jax import lax
from jax.experimental import pallas as pl
from jax.experimental.pallas import tpu as pltpu
from jax.experimental.pallas import tpu_sc as plsc

_N = 10000
_C = 128
_E = 320000
_ALPHA = 0.1
_K = 125
_G = 8
_NGRP = _E // (_K * _G)
_NSC = 2
_NTILE = 16
_GRP_PER_TILE = _NGRP // (_NSC * _NTILE)
_SPAN = 624
_WB = 104
_TAIL_ROW = 16 * _SPAN


def _zero_acc(acc, zbuf, sid, zsem):
    row0 = sid * _SPAN
    ds = []
    for kk in range(_SPAN // _WB):
        ds.append(pltpu.async_copy(
            zbuf, acc.at[pl.ds(row0 + kk * _WB, _WB)], zsem))
    for d in ds:
        d.wait()

    @pl.when(sid == _NTILE - 1)
    def _zero_tail():
        pltpu.sync_copy(zbuf.at[pl.ds(0, 16)], acc.at[pl.ds(_TAIL_ROW, 16)])


def _writeback(acc, zbuf, out_hbm, cid, sid, zsem, wsem):
    row0 = sid * _SPAN
    half = 48
    nck = _SPAN // half
    rd = {}
    wr = {}
    for kk in range(nck):
        r = row0 + kk * half
        z = zbuf.at[pl.ds((kk % 2) * half, half)]
        if kk >= 2:
            wr.pop(kk - 2).wait()
        rd[kk] = pltpu.async_copy(acc.at[pl.ds(r, half)], z, zsem)
        if kk >= 1:
            rd.pop(kk - 1).wait()
            rp = row0 + (kk - 1) * half
            zp = zbuf.at[pl.ds(((kk - 1) % 2) * half, half)]
            wr[kk - 1] = pltpu.async_copy(
                zp, out_hbm.at[cid, pl.ds(rp, half)], wsem)
    rd.pop(nck - 1).wait()
    wr[nck - 1] = pltpu.async_copy(
        zbuf.at[pl.ds(((nck - 1) % 2) * half, half)],
        out_hbm.at[cid, pl.ds(row0 + (nck - 1) * half, half)], wsem)
    wr.pop(nck - 2).wait()
    wr.pop(nck - 1).wait()

    @pl.when(sid == _NTILE - 1)
    def _wb_tail():
        pltpu.sync_copy(acc.at[pl.ds(_TAIL_ROW, 16)], zbuf.at[pl.ds(0, 16)])
        pltpu.sync_copy(zbuf.at[pl.ds(0, 16)],
                        out_hbm.at[cid, pl.ds(_TAIL_ROW, 16)])


def _sc_agg_body(x_hbm, src_hbm, dst_hbm, zeros_hbm,
                 agg_out, acc, buf, isrc, idst, zbuf,
                 gsem0, gsem1, ssem0, ssem1, isem_s, isem_d, zsem, wsem):
    cid = lax.axis_index("c")
    sid = lax.axis_index("s")

    pltpu.sync_copy(zeros_hbm, zbuf)
    _zero_acc(acc, zbuf, sid, zsem)
    plsc.subcore_barrier()

    g0 = (cid * _NTILE + sid) * _GRP_PER_TILE
    gsems = (gsem0, gsem1)
    ssems = (ssem0, ssem1)

    pltpu.sync_copy(src_hbm.at[g0], isrc.at[0])
    pltpu.sync_copy(dst_hbm.at[g0], idst.at[0])

    def group_step(gi, carry):
        slot = gi % 2
        pslot = 1 - slot

        @pl.when(gi > 0)
        def _wait_idx():
            pltpu.make_async_copy(src_hbm.at[g0], isrc.at[slot], isem_s).wait()
            pltpu.make_async_copy(dst_hbm.at[g0], idst.at[slot], isem_d).wait()

        d_g = {}
        d_s = {}
        d_g[0] = pltpu.async_copy(
            x_hbm.at[isrc.at[slot, 0]], buf.at[0], gsems[0])

        @pl.when(gi > 0)
        def _straddle_scatter():
            pltpu.sync_copy(buf.at[1], acc.at[idst.at[pslot, _G - 1]],
                            add=True)

        d_g[1] = pltpu.async_copy(
            x_hbm.at[isrc.at[slot, 1]], buf.at[1], gsems[1])
        d_g.pop(0).wait()
        d_s[0] = pltpu.async_copy(
            buf.at[0], acc.at[idst.at[slot, 0]], ssems[0], add=True)

        for c in range(2, _G):
            d_s.pop(c - 2).wait()
            d_g[c] = pltpu.async_copy(
                x_hbm.at[isrc.at[slot, c]], buf.at[c % 2], gsems[c % 2])
            d_g.pop(c - 1).wait()
            d_s[c - 1] = pltpu.async_copy(
                buf.at[(c - 1) % 2], acc.at[idst.at[slot, c - 1]],
                ssems[(c - 1) % 2], add=True)
            if c == 2:
                @pl.when(gi < _GRP_PER_TILE - 1)
                def _prefetch_idx():
                    pltpu.async_copy(src_hbm.at[g0 + gi + 1],
                                     isrc.at[pslot], isem_s)
                    pltpu.async_copy(dst_hbm.at[g0 + gi + 1],
                                     idst.at[pslot], isem_d)
        d_s.pop(_G - 2).wait()
        d_g.pop(_G - 1).wait()
        return carry

    lax.fori_loop(0, _GRP_PER_TILE, group_step, 0)
    last_slot = (_GRP_PER_TILE - 1) % 2
    pltpu.sync_copy(buf.at[1], acc.at[idst.at[last_slot, _G - 1]], add=True)

    plsc.subcore_barrier()
    _writeback(acc, zbuf, agg_out, cid, sid, zsem, wsem)


_sc_agg = pl.kernel(
    _sc_agg_body,
    out_type=jax.ShapeDtypeStruct((_NSC, _N, _C), jnp.float32),
    mesh=plsc.VectorSubcoreMesh(core_axis_name="c", subcore_axis_name="s"),
    scratch_types=[
        pltpu.VMEM_SHARED((_N, _C), jnp.float32),
        pltpu.VMEM((2, _K, _C), jnp.float32),
        pltpu.VMEM((2, _G, _K), jnp.int32),
        pltpu.VMEM((2, _G, _K), jnp.int32),
        pltpu.VMEM((_WB, _C), jnp.float32),
        pltpu.SemaphoreType.DMA,
        pltpu.SemaphoreType.DMA,
        pltpu.SemaphoreType.DMA,
        pltpu.SemaphoreType.DMA,
        pltpu.SemaphoreType.DMA,
        pltpu.SemaphoreType.DMA,
        pltpu.SemaphoreType.DMA,
        pltpu.SemaphoreType.DMA,
    ],
)


def _sc_deg_body(dst_hbm, zeros_hbm, ones_hbm,
                 deg_out, acc, ones_v, zbuf, idst,
                 ssem0, ssem1, isem_d, zsem, wsem):
    cid = lax.axis_index("c")
    sid = lax.axis_index("s")

    pltpu.sync_copy(zeros_hbm, zbuf)
    pltpu.sync_copy(ones_hbm, ones_v)
    _zero_acc(acc, zbuf, sid, zsem)
    plsc.subcore_barrier()

    g0 = (cid * _NTILE + sid) * _GRP_PER_TILE
    ssems = (ssem0, ssem1)

    pltpu.sync_copy(dst_hbm.at[g0], idst.at[0])

    def group_step(gi, carry):
        slot = gi % 2
        pslot = 1 - slot

        @pl.when(gi > 0)
        def _wait_idx():
            pltpu.make_async_copy(dst_hbm.at[g0], idst.at[slot], isem_d).wait()

        d_s = {}
        for c in range(_G):
            if c >= 2:
                d_s.pop(c - 2).wait()
            d_s[c] = pltpu.async_copy(
                ones_v, acc.at[idst.at[slot, c]], ssems[c % 2], add=True)
            if c == 2:
                @pl.when(gi < _GRP_PER_TILE - 1)
                def _prefetch_idx():
                    pltpu.async_copy(dst_hbm.at[g0 + gi + 1],
                                     idst.at[pslot], isem_d)
        d_s.pop(_G - 2).wait()
        d_s.pop(_G - 1).wait()
        return carry

    lax.fori_loop(0, _GRP_PER_TILE, group_step, 0)
    plsc.subcore_barrier()
    _writeback(acc, zbuf, deg_out, cid, sid, zsem, wsem)


_sc_deg = pl.kernel(
    _sc_deg_body,
    out_type=jax.ShapeDtypeStruct((_NSC, _N, _C), jnp.float32),
    mesh=plsc.VectorSubcoreMesh(core_axis_name="c", subcore_axis_name="s"),
    scratch_types=[
        pltpu.VMEM_SHARED((_N, _C), jnp.float32),
        pltpu.VMEM((_K, _C), jnp.float32),
        pltpu.VMEM((_WB, _C), jnp.float32),
        pltpu.VMEM((2, _G, _K), jnp.int32),
        pltpu.SemaphoreType.DMA,
        pltpu.SemaphoreType.DMA,
        pltpu.SemaphoreType.DMA,
        pltpu.SemaphoreType.DMA,
        pltpu.SemaphoreType.DMA,
    ],
)


def _mlp_body(x_ref, w_ref, g_ref, b_ref, o_ref):
    h = jnp.dot(x_ref[...], w_ref[...], preferred_element_type=jnp.float32)
    mu = jnp.mean(h, axis=0, keepdims=True)
    var = jnp.mean((h - mu) ** 2, axis=0, keepdims=True)
    hn = (h - mu) * lax.rsqrt(var + 1e-5)
    o_ref[...] = jnp.maximum(g_ref[...] * hn + b_ref[...], 0.0)


def _tc_mlp(x, w, gamma, beta):
    return pl.pallas_call(
        _mlp_body,
        out_shape=jax.ShapeDtypeStruct((_N, _C), jnp.float32),
    )(x, w, gamma.reshape(1, _C), beta.reshape(1, _C))


_BN = 1000


def _fin0_body(pdeg_ref, pa0_ref, pa1_ref, inv_ref, a0_ref, a1_ref):
    deg = pdeg_ref[0] + pdeg_ref[1]
    inv = 1.0 / jnp.maximum(deg, 1.0)
    inv_ref[...] = inv[:, 0:1]
    a0_ref[...] = (pa0_ref[0] + pa0_ref[1]) * inv
    a1_ref[...] = (pa1_ref[0] + pa1_ref[1]) * inv


def _tc_finalize0(pdeg, pa0, pa1):
    grid = (_N // _BN,)
    bs_p = pl.BlockSpec((_NSC, _BN, _C), lambda i: (0, i, 0))
    bs_o = pl.BlockSpec((_BN, _C), lambda i: (i, 0))
    bs_i = pl.BlockSpec((_BN, 1), lambda i: (i, 0))
    return pl.pallas_call(
        _fin0_body,
        grid=grid,
        in_specs=[bs_p, bs_p, bs_p],
        out_specs=[bs_i, bs_o, bs_o],
        out_shape=[
            jax.ShapeDtypeStruct((_N, 1), jnp.float32),
            jax.ShapeDtypeStruct((_N, _C), jnp.float32),
            jax.ShapeDtypeStruct((_N, _C), jnp.float32),
        ],
    )(pdeg, pa0, pa1)


def _fin_body(pa_ref, inv_ref, a_ref):
    a_ref[...] = (pa_ref[0] + pa_ref[1]) * inv_ref[...]


def _tc_finalize(pa, inv_deg):
    grid = (_N // _BN,)
    return pl.pallas_call(
        _fin_body,
        grid=grid,
        in_specs=[
            pl.BlockSpec((_NSC, _BN, _C), lambda i: (0, i, 0)),
            pl.BlockSpec((_BN, 1), lambda i: (i, 0)),
        ],
        out_specs=pl.BlockSpec((_BN, _C), lambda i: (i, 0)),
        out_shape=jax.ShapeDtypeStruct((_N, _C), jnp.float32),
    )(pa, inv_deg)


def _combine_body(k, wrow_ref, x0_ref, *refs):
    hs = refs[:k]
    asm = refs[k:2 * k]
    wg_ref, wss_ref, wsn_ref, o_ref = refs[2 * k:]
    x0 = x0_ref[...]
    acc = jnp.zeros_like(x0)
    for j in range(k):
        h = hs[j][...]
        a = asm[j][...]
        w1 = wrow_ref[j, 1]
        w2 = wrow_ref[j, 2]
        w3 = wrow_ref[j, 3]
        u = (1.0 - _ALPHA) * a + _ALPHA * x0
        gcn = jnp.maximum(
            jnp.dot(u, wg_ref[j], preferred_element_type=jnp.float32), 0.0)
        sage = jnp.maximum(
            jnp.dot(h, wss_ref[j], preferred_element_type=jnp.float32)
            + jnp.dot(a, wsn_ref[j], preferred_element_type=jnp.float32), 0.0)
        acc = acc + w1 * h + w2 * gcn + w3 * sage
    o_ref[...] = acc


def _tc_combine(k, wrow, x0, hs, ams, wg, wss, wsn):
    grid = (_N // _BN,)
    bs_row = pl.BlockSpec((_BN, _C), lambda i: (i, 0))
    bs_w = pl.BlockSpec((k, _C, _C), lambda i: (0, 0, 0))
    return pl.pallas_call(
        functools.partial(_combine_body, k),
        grid=grid,
        in_specs=([pl.BlockSpec(memory_space=pltpu.SMEM), bs_row]
                  + [bs_row] * (2 * k) + [bs_w, bs_w, bs_w]),
        out_specs=bs_row,
        out_shape=jax.ShapeDtypeStruct((_N, _C), jnp.float32),
    )(wrow, x0, *hs, *ams, wg, wss, wsn)


def kernel(s0, s1, x_0, edge_index, weights, W0, gamma0, beta0,
           W1, gamma1, beta1, Wg, Wss, Wsn):
    src3d = edge_index[0].astype(jnp.int32).reshape(_NGRP, _G, _K)
    dst3d = edge_index[1].astype(jnp.int32).reshape(_NGRP, _G, _K)
    zeros_c = jnp.zeros((_WB, _C), jnp.float32)
    ones_c = jnp.ones((_K, _C), jnp.float32)

    m0 = _tc_mlp(s0, W0, gamma0, beta0)
    m1 = _tc_mlp(s1, W1, gamma1, beta1)

    def agg(x):
        return _sc_agg(x, src3d, dst3d, zeros_c)

    pdeg = _sc_deg(dst3d, zeros_c, ones_c)
    pa0 = agg(m0)
    pa1 = agg(m1)
    inv_deg, a0, a1 = _tc_finalize0(pdeg, pa0, pa1)

    states = [m0, m1]
    aggs = [a0, a1]
    offset = 0
    for step in range(4):
        k = len(states)
        sl = slice(offset, offset + k)
        new_state = _tc_combine(k, weights[sl], x_0, states, aggs,
                                Wg[sl], Wss[sl], Wsn[sl])
        offset += k
        states.append(new_state)
        if step < 3:
            aggs.append(_tc_finalize(agg(new_state), inv_deg))

    return jnp.concatenate(states[-4:], axis=1)


# speedup vs baseline: 10.8254x; 1.0417x over previous
import functools

import jax
import jax.numpy as jnp
from jax import lax
from jax.experimental import pallas as pl
from jax.experimental.pallas import tpu as pltpu
from jax.experimental.pallas import tpu_sc as plsc

_N = 10000
_C = 128
_E = 320000
_ALPHA = 0.1
_K = 125
_G = 8
_NGRP = _E // (_K * _G)
_NSC = 2
_NTILE = 16
_GRP_PER_TILE = _NGRP // (_NSC * _NTILE)
_SPAN = 624
_WB = 104
_TAIL_ROW = 16 * _SPAN


def _zero_acc(acc, zbuf, sid, zsem):
    row0 = sid * _SPAN
    ds = []
    for kk in range(_SPAN // _WB):
        ds.append(pltpu.async_copy(
            zbuf, acc.at[pl.ds(row0 + kk * _WB, _WB)], zsem))
    for d in ds:
        d.wait()

    @pl.when(sid == _NTILE - 1)
    def _zero_tail():
        pltpu.sync_copy(zbuf.at[pl.ds(0, 16)], acc.at[pl.ds(_TAIL_ROW, 16)])


def _writeback(acc, zbuf, out_hbm, cid, sid, zsem, wsem):
    row0 = sid * _SPAN
    half = 48
    nck = _SPAN // half
    rd = {}
    wr = {}
    for kk in range(nck):
        r = row0 + kk * half
        z = zbuf.at[pl.ds((kk % 2) * half, half)]
        if kk >= 2:
            wr.pop(kk - 2).wait()
        rd[kk] = pltpu.async_copy(acc.at[pl.ds(r, half)], z, zsem)
        if kk >= 1:
            rd.pop(kk - 1).wait()
            rp = row0 + (kk - 1) * half
            zp = zbuf.at[pl.ds(((kk - 1) % 2) * half, half)]
            wr[kk - 1] = pltpu.async_copy(
                zp, out_hbm.at[cid, pl.ds(rp, half)], wsem)
    rd.pop(nck - 1).wait()
    wr[nck - 1] = pltpu.async_copy(
        zbuf.at[pl.ds(((nck - 1) % 2) * half, half)],
        out_hbm.at[cid, pl.ds(row0 + (nck - 1) * half, half)], wsem)
    wr.pop(nck - 2).wait()
    wr.pop(nck - 1).wait()

    @pl.when(sid == _NTILE - 1)
    def _wb_tail():
        pltpu.sync_copy(acc.at[pl.ds(_TAIL_ROW, 16)], zbuf.at[pl.ds(0, 16)])
        pltpu.sync_copy(zbuf.at[pl.ds(0, 16)],
                        out_hbm.at[cid, pl.ds(_TAIL_ROW, 16)])


def _sc_agg_body(x_hbm, src_hbm, dst_hbm, zeros_hbm,
                 agg_out, acc, buf, isrc, idst, zbuf,
                 gsem0, gsem1, ssem0, ssem1, isem_s, isem_d, zsem, wsem):
    cid = lax.axis_index("c")
    sid = lax.axis_index("s")

    pltpu.sync_copy(zeros_hbm, zbuf)
    _zero_acc(acc, zbuf, sid, zsem)
    plsc.subcore_barrier()

    g0 = (cid * _NTILE + sid) * _GRP_PER_TILE
    gsems = (gsem0, gsem1)
    ssems = (ssem0, ssem1)

    pltpu.sync_copy(src_hbm.at[g0], isrc.at[0])
    pltpu.sync_copy(dst_hbm.at[g0], idst.at[0])

    def group_step(gi, carry):
        slot = gi % 2
        pslot = 1 - slot

        @pl.when(gi > 0)
        def _wait_idx():
            pltpu.make_async_copy(src_hbm.at[g0], isrc.at[slot], isem_s).wait()
            pltpu.make_async_copy(dst_hbm.at[g0], idst.at[slot], isem_d).wait()

        d_g = {}
        d_s = {}
        d_g[0] = pltpu.async_copy(
            x_hbm.at[isrc.at[slot, 0]], buf.at[0], gsems[0])

        @pl.when(gi > 0)
        def _straddle_scatter():
            pltpu.sync_copy(buf.at[1], acc.at[idst.at[pslot, _G - 1]],
                            add=True)

        d_g[1] = pltpu.async_copy(
            x_hbm.at[isrc.at[slot, 1]], buf.at[1], gsems[1])
        d_g.pop(0).wait()
        d_s[0] = pltpu.async_copy(
            buf.at[0], acc.at[idst.at[slot, 0]], ssems[0], add=True)

        for c in range(2, _G):
            d_s.pop(c - 2).wait()
            d_g[c] = pltpu.async_copy(
                x_hbm.at[isrc.at[slot, c]], buf.at[c % 2], gsems[c % 2])
            d_g.pop(c - 1).wait()
            d_s[c - 1] = pltpu.async_copy(
                buf.at[(c - 1) % 2], acc.at[idst.at[slot, c - 1]],
                ssems[(c - 1) % 2], add=True)
            if c == 2:
                @pl.when(gi < _GRP_PER_TILE - 1)
                def _prefetch_idx():
                    pltpu.async_copy(src_hbm.at[g0 + gi + 1],
                                     isrc.at[pslot], isem_s)
                    pltpu.async_copy(dst_hbm.at[g0 + gi + 1],
                                     idst.at[pslot], isem_d)
        d_s.pop(_G - 2).wait()
        d_g.pop(_G - 1).wait()
        return carry

    lax.fori_loop(0, _GRP_PER_TILE, group_step, 0)
    last_slot = (_GRP_PER_TILE - 1) % 2
    pltpu.sync_copy(buf.at[1], acc.at[idst.at[last_slot, _G - 1]], add=True)

    plsc.subcore_barrier()
    _writeback(acc, zbuf, agg_out, cid, sid, zsem, wsem)


_sc_agg = pl.kernel(
    _sc_agg_body,
    out_type=jax.ShapeDtypeStruct((_NSC, _N, _C), jnp.float32),
    mesh=plsc.VectorSubcoreMesh(core_axis_name="c", subcore_axis_name="s"),
    scratch_types=[
        pltpu.VMEM_SHARED((_N, _C), jnp.float32),
        pltpu.VMEM((2, _K, _C), jnp.float32),
        pltpu.VMEM((2, _G, _K), jnp.int32),
        pltpu.VMEM((2, _G, _K), jnp.int32),
        pltpu.VMEM((_WB, _C), jnp.float32),
        pltpu.SemaphoreType.DMA,
        pltpu.SemaphoreType.DMA,
        pltpu.SemaphoreType.DMA,
        pltpu.SemaphoreType.DMA,
        pltpu.SemaphoreType.DMA,
        pltpu.SemaphoreType.DMA,
        pltpu.SemaphoreType.DMA,
        pltpu.SemaphoreType.DMA,
    ],
)


def _sc_deg_body(dst_hbm, zeros_hbm, ones_hbm,
                 deg_out, acc, ones_v, zbuf, idst,
                 ssem0, ssem1, isem_d, zsem, wsem):
    cid = lax.axis_index("c")
    sid = lax.axis_index("s")

    pltpu.sync_copy(zeros_hbm, zbuf)
    pltpu.sync_copy(ones_hbm, ones_v)
    _zero_acc(acc, zbuf, sid, zsem)
    plsc.subcore_barrier()

    g0 = (cid * _NTILE + sid) * _GRP_PER_TILE
    ssems = (ssem0, ssem1)

    pltpu.sync_copy(dst_hbm.at[g0], idst.at[0])

    def group_step(gi, carry):
        slot = gi % 2
        pslot = 1 - slot

        @pl.when(gi > 0)
        def _wait_idx():
            pltpu.make_async_copy(dst_hbm.at[g0], idst.at[slot], isem_d).wait()

        d_s = {}
        for c in range(_G):
            if c >= 2:
                d_s.pop(c - 2).wait()
            d_s[c] = pltpu.async_copy(
                ones_v, acc.at[idst.at[slot, c]], ssems[c % 2], add=True)
            if c == 2:
                @pl.when(gi < _GRP_PER_TILE - 1)
                def _prefetch_idx():
                    pltpu.async_copy(dst_hbm.at[g0 + gi + 1],
                                     idst.at[pslot], isem_d)
        d_s.pop(_G - 2).wait()
        d_s.pop(_G - 1).wait()
        return carry

    lax.fori_loop(0, _GRP_PER_TILE, group_step, 0)
    plsc.subcore_barrier()
    _writeback(acc, zbuf, deg_out, cid, sid, zsem, wsem)


_sc_deg = pl.kernel(
    _sc_deg_body,
    out_type=jax.ShapeDtypeStruct((_NSC, _N, _C), jnp.float32),
    mesh=plsc.VectorSubcoreMesh(core_axis_name="c", subcore_axis_name="s"),
    scratch_types=[
        pltpu.VMEM_SHARED((_N, _C), jnp.float32),
        pltpu.VMEM((_K, _C), jnp.float32),
        pltpu.VMEM((_WB, _C), jnp.float32),
        pltpu.VMEM((2, _G, _K), jnp.int32),
        pltpu.SemaphoreType.DMA,
        pltpu.SemaphoreType.DMA,
        pltpu.SemaphoreType.DMA,
        pltpu.SemaphoreType.DMA,
        pltpu.SemaphoreType.DMA,
    ],
)


def _mlp_body(x0_ref, w0_ref, g0_ref, b0_ref, x1_ref, w1_ref, g1_ref, b1_ref,
              o0_ref, o1_ref):
    for x_ref, w_ref, g_ref, b_ref, o_ref in (
            (x0_ref, w0_ref, g0_ref, b0_ref, o0_ref),
            (x1_ref, w1_ref, g1_ref, b1_ref, o1_ref)):
        h = jnp.dot(x_ref[...], w_ref[...], preferred_element_type=jnp.float32)
        mu = jnp.mean(h, axis=0, keepdims=True)
        var = jnp.mean((h - mu) ** 2, axis=0, keepdims=True)
        hn = (h - mu) * lax.rsqrt(var + 1e-5)
        o_ref[...] = jnp.maximum(g_ref[...] * hn + b_ref[...], 0.0)


def _tc_mlp2(s0, w0, gamma0, beta0, s1, w1, gamma1, beta1):
    return pl.pallas_call(
        _mlp_body,
        out_shape=[jax.ShapeDtypeStruct((_N, _C), jnp.float32),
                   jax.ShapeDtypeStruct((_N, _C), jnp.float32)],
    )(s0, w0, gamma0.reshape(1, _C), beta0.reshape(1, _C),
      s1, w1, gamma1.reshape(1, _C), beta1.reshape(1, _C))


_BN = 1000


def _edge_term(wrow_ref, j, h, a, x0, wg_ref, wss_ref, wsn_ref):
    w1 = wrow_ref[j, 1]
    w2 = wrow_ref[j, 2]
    w3 = wrow_ref[j, 3]
    u = (1.0 - _ALPHA) * a + _ALPHA * x0
    gcn = jnp.maximum(
        jnp.dot(u, wg_ref[j], preferred_element_type=jnp.float32), 0.0)
    sage = jnp.maximum(
        jnp.dot(h, wss_ref[j], preferred_element_type=jnp.float32)
        + jnp.dot(a, wsn_ref[j], preferred_element_type=jnp.float32), 0.0)
    return w1 * h + w2 * gcn + w3 * sage


def _combine0_body(wrow_ref, x0_ref, h0_ref, h1_ref, pdeg_ref, pa0_ref,
                   pa1_ref, wg_ref, wss_ref, wsn_ref,
                   o_ref, inv_ref, a0_ref, a1_ref):
    inv = 1.0 / jnp.maximum(pdeg_ref[0] + pdeg_ref[1], 1.0)
    inv_ref[...] = inv[:, 0:1]
    a0 = (pa0_ref[0] + pa0_ref[1]) * inv
    a1 = (pa1_ref[0] + pa1_ref[1]) * inv
    a0_ref[...] = a0
    a1_ref[...] = a1
    x0 = x0_ref[...]
    o_ref[...] = (
        _edge_term(wrow_ref, 0, h0_ref[...], a0, x0, wg_ref, wss_ref, wsn_ref)
        + _edge_term(wrow_ref, 1, h1_ref[...], a1, x0, wg_ref, wss_ref,
                     wsn_ref))


def _tc_combine0(wrow, x0, h0, h1, pdeg, pa0, pa1, wg, wss, wsn):
    grid = (_N // _BN,)
    bs_row = pl.BlockSpec((_BN, _C), lambda i: (i, 0))
    bs_p = pl.BlockSpec((_NSC, _BN, _C), lambda i: (0, i, 0))
    bs_w = pl.BlockSpec((2, _C, _C), lambda i: (0, 0, 0))
    return pl.pallas_call(
        _combine0_body,
        grid=grid,
        in_specs=[pl.BlockSpec(memory_space=pltpu.SMEM), bs_row, bs_row,
                  bs_row, bs_p, bs_p, bs_p, bs_w, bs_w, bs_w],
        out_specs=[bs_row, pl.BlockSpec((_BN, 1), lambda i: (i, 0)),
                   bs_row, bs_row],
        out_shape=[
            jax.ShapeDtypeStruct((_N, _C), jnp.float32),
            jax.ShapeDtypeStruct((_N, 1), jnp.float32),
            jax.ShapeDtypeStruct((_N, _C), jnp.float32),
            jax.ShapeDtypeStruct((_N, _C), jnp.float32),
        ],
    )(wrow, x0, h0, h1, pdeg, pa0, pa1, wg, wss, wsn)


def _combine_body(k, wrow_ref, x0_ref, inv_ref, *refs):
    hs = refs[:k]
    ams = refs[k:2 * k - 1]
    pa_ref, wg_ref, wss_ref, wsn_ref, o_ref, anew_ref = refs[2 * k - 1:]
    x0 = x0_ref[...]
    anew = (pa_ref[0] + pa_ref[1]) * inv_ref[...]
    anew_ref[...] = anew
    acc = _edge_term(wrow_ref, k - 1, hs[k - 1][...], anew, x0,
                     wg_ref, wss_ref, wsn_ref)
    for j in range(k - 1):
        acc = acc + _edge_term(wrow_ref, j, hs[j][...], ams[j][...], x0,
                               wg_ref, wss_ref, wsn_ref)
    o_ref[...] = acc


def _tc_combine(k, wrow, x0, inv_deg, hs, ams, pa_new, wg, wss, wsn):
    grid = (_N // _BN,)
    bs_row = pl.BlockSpec((_BN, _C), lambda i: (i, 0))
    bs_p = pl.BlockSpec((_NSC, _BN, _C), lambda i: (0, i, 0))
    bs_w = pl.BlockSpec((k, _C, _C), lambda i: (0, 0, 0))
    bs_i = pl.BlockSpec((_BN, 1), lambda i: (i, 0))
    return pl.pallas_call(
        functools.partial(_combine_body, k),
        grid=grid,
        in_specs=([pl.BlockSpec(memory_space=pltpu.SMEM), bs_row, bs_i]
                  + [bs_row] * (2 * k - 1) + [bs_p, bs_w, bs_w, bs_w]),
        out_specs=[bs_row, bs_row],
        out_shape=[jax.ShapeDtypeStruct((_N, _C), jnp.float32),
                   jax.ShapeDtypeStruct((_N, _C), jnp.float32)],
    )(wrow, x0, inv_deg, *hs, *ams, pa_new, wg, wss, wsn)


def kernel(s0, s1, x_0, edge_index, weights, W0, gamma0, beta0,
           W1, gamma1, beta1, Wg, Wss, Wsn):
    src3d = edge_index[0].astype(jnp.int32).reshape(_NGRP, _G, _K)
    dst3d = edge_index[1].astype(jnp.int32).reshape(_NGRP, _G, _K)
    zeros_c = jnp.zeros((_WB, _C), jnp.float32)
    ones_c = jnp.ones((_K, _C), jnp.float32)

    pdeg = _sc_deg(dst3d, zeros_c, ones_c)
    m0, m1 = _tc_mlp2(s0, W0, gamma0, beta0, s1, W1, gamma1, beta1)

    def agg(x):
        return _sc_agg(x, src3d, dst3d, zeros_c)

    pa0 = agg(m0)
    pa1 = agg(m1)

    s2, inv_deg, a0, a1 = _tc_combine0(
        weights[0:2], x_0, m0, m1, pdeg, pa0, pa1,
        Wg[0:2], Wss[0:2], Wsn[0:2])

    states = [m0, m1, s2]
    aggs = [a0, a1]
    offset = 2
    for step in range(1, 4):
        k = len(states)
        sl = slice(offset, offset + k)
        pa_new = agg(states[-1])
        new_state, a_new = _tc_combine(
            k, weights[sl], x_0, inv_deg, states, aggs, pa_new,
            Wg[sl], Wss[sl], Wsn[sl])
        offset += k
        states.append(new_state)
        aggs.append(a_new)

    return jnp.concatenate(states[-4:], axis=1)
